# baseline (device time: 44900 ns/iter reference)
import jax
import jax.numpy as jnp
from jax import lax
from jax.experimental import pallas as pl
from jax.experimental.pallas import tpu as pltpu

N_Z = 4
P = 4
SCALE = 127.0 / 3.0
INV_SCALE = 3.0 / 127.0


def kernel(x, W):
    m, _ = x.shape
    n_local = W.shape[1]
    n_half = n_local // 2
    n_piece = n_half // P

    def body(x_ref, w_ref, out_ref, comm_z, comm_x, z_send, z_recv,
             x_send, x_recv):
        my_x = lax.axis_index("x")
        my_y = lax.axis_index("y")
        my_z = lax.axis_index("z")
        left = (my_z - 1) % N_Z
        right = (my_z + 1) % N_Z

        zdescs = [
            [
                pltpu.make_async_remote_copy(
                    src_ref=comm_z.at[h, p],
                    dst_ref=comm_z.at[h + 1, p],
                    send_sem=z_send.at[h, p],
                    recv_sem=z_recv.at[h, p],
                    device_id=(my_x, my_y, right),
                    device_id_type=pl.DeviceIdType.MESH,
                )
                for p in range(P)
            ]
            for h in range(N_Z - 1)
        ]
        xdescs = [
            [
                pltpu.make_async_remote_copy(
                    src_ref=comm_z.at[h + 1, p],
                    dst_ref=comm_x.at[h, p],
                    send_sem=x_send.at[h, p],
                    recv_sem=x_recv.at[h, p],
                    device_id=(1 - my_x, my_y, my_z),
                    device_id_type=pl.DeviceIdType.MESH,
                )
                for p in range(P)
            ]
            for h in range(N_Z - 1)
        ]

        barrier_sem = pltpu.get_barrier_semaphore()
        for dev in ((my_x, my_y, left), (my_x, my_y, right),
                    (1 - my_x, my_y, my_z)):
            pl.semaphore_signal(
                barrier_sem, inc=1,
                device_id=dev,
                device_id_type=pl.DeviceIdType.MESH,
            )
        pl.semaphore_wait(barrier_sem, 3)

        x_bf = x_ref[...].astype(jnp.bfloat16)

        mine_pieces = []
        for p in range(P):
            w_p = w_ref[:, pl.ds(my_x * n_half + p * n_piece, n_piece)].astype(
                jnp.bfloat16
            )
            l_p = jnp.dot(x_bf, w_p, preferred_element_type=jnp.float32)
            comm_z[0, p] = jnp.clip(
                jnp.round(l_p * SCALE), -127.0, 127.0
            ).astype(jnp.int8)
            zdescs[0][p].start()
            mine_pieces.append(l_p)

        w_other = w_ref[:, pl.ds((1 - my_x) * n_half, n_half)].astype(
            jnp.bfloat16
        )
        logits_other = jnp.dot(
            x_bf, w_other, preferred_element_type=jnp.float32
        )
        e_other = jnp.exp(logits_other)
        s = jnp.sum(e_other, axis=-1, keepdims=True)
        out_ref[:, pl.ds(my_z * n_local + (1 - my_x) * n_half, n_half)] = (
            e_other.astype(jnp.bfloat16)
        )
        for p in range(P):
            e_p = jnp.exp(mine_pieces[p])
            s = s + jnp.sum(e_p, axis=-1, keepdims=True)
            out_ref[
                :, pl.ds(my_z * n_local + my_x * n_half + p * n_piece, n_piece)
            ] = e_p.astype(jnp.bfloat16)

        seq = [(h, p) for h in range(1, N_Z) for p in range(P)]
        prev = None
        for h, p in seq:
            zdescs[h - 1][p].wait_recv()
            if h < N_Z - 1:
                zdescs[h][p].start()
            xdescs[h - 1][p].start()

            origin = (my_z - h) % N_Z
            e_z = jnp.exp(comm_z[h, p].astype(jnp.float32) * INV_SCALE)
            s = s + jnp.sum(e_z, axis=-1, keepdims=True)
            col = origin * n_local + my_x * n_half + p * n_piece
            out_ref[:, pl.ds(col, n_piece)] = e_z.astype(jnp.bfloat16)

            if prev is not None:
                hh, pp = prev
                xdescs[hh - 1][pp].wait_recv()
                origin_x = (my_z - hh) % N_Z
                e_x = jnp.exp(
                    comm_x[hh - 1, pp].astype(jnp.float32) * INV_SCALE
                )
                s = s + jnp.sum(e_x, axis=-1, keepdims=True)
                col_x = origin_x * n_local + (1 - my_x) * n_half + pp * n_piece
                out_ref[:, pl.ds(col_x, n_piece)] = e_x.astype(jnp.bfloat16)
            prev = (h, p)

        hh, pp = prev
        xdescs[hh - 1][pp].wait_recv()
        origin_x = (my_z - hh) % N_Z
        e_x = jnp.exp(comm_x[hh - 1, pp].astype(jnp.float32) * INV_SCALE)
        s = s + jnp.sum(e_x, axis=-1, keepdims=True)
        col_x = origin_x * n_local + (1 - my_x) * n_half + pp * n_piece
        out_ref[:, pl.ds(col_x, n_piece)] = e_x.astype(jnp.bfloat16)

        for row in zdescs + xdescs:
            for d in row:
                d.wait_send()

        inv = (1.0 / s).astype(jnp.bfloat16)
        out_ref[...] = out_ref[...] * inv

    return pl.pallas_call(
        body,
        out_shape=jax.ShapeDtypeStruct((m, N_Z * n_local), jnp.bfloat16),
        in_specs=[
            pl.BlockSpec(memory_space=pltpu.VMEM),
            pl.BlockSpec(memory_space=pltpu.VMEM),
        ],
        out_specs=pl.BlockSpec(memory_space=pltpu.VMEM),
        scratch_shapes=[
            pltpu.VMEM((N_Z, P, m, n_piece), jnp.int8),
            pltpu.VMEM((N_Z - 1, P, m, n_piece), jnp.int8),
            pltpu.SemaphoreType.DMA((N_Z - 1, P)),
            pltpu.SemaphoreType.DMA((N_Z - 1, P)),
            pltpu.SemaphoreType.DMA((N_Z - 1, P)),
            pltpu.SemaphoreType.DMA((N_Z - 1, P)),
        ],
        compiler_params=pltpu.CompilerParams(
            collective_id=0,
            vmem_limit_bytes=100 * 1024 * 1024,
        ),
    )(x, W)


# device time: 43675 ns/iter; 1.0280x vs baseline; 1.0280x over previous
import jax
import jax.numpy as jnp
from jax import lax
from jax.experimental import pallas as pl
from jax.experimental.pallas import tpu as pltpu

N_Z = 4
P = 2
SCALE = 127.0 / 3.0
INV_SCALE = 3.0 / 127.0


def kernel(x, W):
    m, _ = x.shape
    n_local = W.shape[1]
    n_half = n_local // 2
    n_piece = n_half // P

    def body(x_ref, w_ref, out_ref, comm_z, comm_x, z_send, z_recv,
             x_send, x_recv):
        my_x = lax.axis_index("x")
        my_y = lax.axis_index("y")
        my_z = lax.axis_index("z")
        left = (my_z - 1) % N_Z
        right = (my_z + 1) % N_Z

        zdescs = [
            [
                pltpu.make_async_remote_copy(
                    src_ref=comm_z.at[h, p],
                    dst_ref=comm_z.at[h + 1, p],
                    send_sem=z_send.at[h, p],
                    recv_sem=z_recv.at[h, p],
                    device_id=(my_x, my_y, right),
                    device_id_type=pl.DeviceIdType.MESH,
                )
                for p in range(P)
            ]
            for h in range(N_Z - 1)
        ]
        xdescs = [
            [
                pltpu.make_async_remote_copy(
                    src_ref=comm_z.at[h + 1, p],
                    dst_ref=comm_x.at[h, p],
                    send_sem=x_send.at[h, p],
                    recv_sem=x_recv.at[h, p],
                    device_id=(1 - my_x, my_y, my_z),
                    device_id_type=pl.DeviceIdType.MESH,
                )
                for p in range(P)
            ]
            for h in range(N_Z - 1)
        ]

        barrier_sem = pltpu.get_barrier_semaphore()
        for dev in ((my_x, my_y, left), (my_x, my_y, right),
                    (1 - my_x, my_y, my_z)):
            pl.semaphore_signal(
                barrier_sem, inc=1,
                device_id=dev,
                device_id_type=pl.DeviceIdType.MESH,
            )
        pl.semaphore_wait(barrier_sem, 3)

        x_bf = x_ref[...].astype(jnp.bfloat16)

        mine_pieces = []
        for p in range(P):
            w_p = w_ref[:, pl.ds(my_x * n_half + p * n_piece, n_piece)].astype(
                jnp.bfloat16
            )
            l_p = jnp.dot(x_bf, w_p, preferred_element_type=jnp.float32)
            comm_z[0, p] = jnp.clip(
                jnp.round(l_p * SCALE), -127.0, 127.0
            ).astype(jnp.int8)
            zdescs[0][p].start()
            mine_pieces.append(l_p)

        w_other = w_ref[:, pl.ds((1 - my_x) * n_half, n_half)].astype(
            jnp.bfloat16
        )
        logits_other = jnp.dot(
            x_bf, w_other, preferred_element_type=jnp.float32
        )
        e_other = jnp.exp(logits_other)
        s = jnp.sum(e_other, axis=-1, keepdims=True)
        out_ref[:, pl.ds(my_z * n_local + (1 - my_x) * n_half, n_half)] = (
            e_other.astype(jnp.bfloat16)
        )
        for p in range(P):
            e_p = jnp.exp(mine_pieces[p])
            s = s + jnp.sum(e_p, axis=-1, keepdims=True)
            out_ref[
                :, pl.ds(my_z * n_local + my_x * n_half + p * n_piece, n_piece)
            ] = e_p.astype(jnp.bfloat16)

        seq = [(h, p) for h in range(1, N_Z) for p in range(P)]
        prev = None
        for h, p in seq:
            zdescs[h - 1][p].wait_recv()
            if h < N_Z - 1:
                zdescs[h][p].start()
            xdescs[h - 1][p].start()

            origin = (my_z - h) % N_Z
            e_z = jnp.exp(comm_z[h, p].astype(jnp.float32) * INV_SCALE)
            s = s + jnp.sum(e_z, axis=-1, keepdims=True)
            col = origin * n_local + my_x * n_half + p * n_piece
            out_ref[:, pl.ds(col, n_piece)] = e_z.astype(jnp.bfloat16)

            if prev is not None:
                hh, pp = prev
                xdescs[hh - 1][pp].wait_recv()
                origin_x = (my_z - hh) % N_Z
                e_x = jnp.exp(
                    comm_x[hh - 1, pp].astype(jnp.float32) * INV_SCALE
                )
                s = s + jnp.sum(e_x, axis=-1, keepdims=True)
                col_x = origin_x * n_local + (1 - my_x) * n_half + pp * n_piece
                out_ref[:, pl.ds(col_x, n_piece)] = e_x.astype(jnp.bfloat16)
            prev = (h, p)

        hh, pp = prev
        xdescs[hh - 1][pp].wait_recv()
        origin_x = (my_z - hh) % N_Z
        e_x = jnp.exp(comm_x[hh - 1, pp].astype(jnp.float32) * INV_SCALE)
        s = s + jnp.sum(e_x, axis=-1, keepdims=True)
        col_x = origin_x * n_local + (1 - my_x) * n_half + pp * n_piece
        out_ref[:, pl.ds(col_x, n_piece)] = e_x.astype(jnp.bfloat16)

        for row in zdescs + xdescs:
            for d in row:
                d.wait_send()

        inv = (1.0 / s).astype(jnp.bfloat16)
        out_ref[...] = out_ref[...] * inv

    return pl.pallas_call(
        body,
        out_shape=jax.ShapeDtypeStruct((m, N_Z * n_local), jnp.bfloat16),
        in_specs=[
            pl.BlockSpec(memory_space=pltpu.VMEM),
            pl.BlockSpec(memory_space=pltpu.VMEM),
        ],
        out_specs=pl.BlockSpec(memory_space=pltpu.VMEM),
        scratch_shapes=[
            pltpu.VMEM((N_Z, P, m, n_piece), jnp.int8),
            pltpu.VMEM((N_Z - 1, P, m, n_piece), jnp.int8),
            pltpu.SemaphoreType.DMA((N_Z - 1, P)),
            pltpu.SemaphoreType.DMA((N_Z - 1, P)),
            pltpu.SemaphoreType.DMA((N_Z - 1, P)),
            pltpu.SemaphoreType.DMA((N_Z - 1, P)),
        ],
        compiler_params=pltpu.CompilerParams(
            collective_id=0,
            vmem_limit_bytes=100 * 1024 * 1024,
        ),
    )(x, W)
